# Initial kernel scaffold; baseline (speedup 1.0000x reference)
import functools

import jax
import jax.numpy as jnp
from jax import lax
from jax.experimental import pallas as pl
from jax.experimental.pallas import tpu as pltpu
from jax.experimental.pallas import tpu_sc as plsc

N = 10000
E = 320000
D = 128
H = 128
G = 64
C = 10

NC = 2
NS = 16
NW = NC * NS

CH = 128
CPT = -(-(E // NW) // CH)
E_PAD = NW * CPT * CH
N_ACC = 10240
ZCH = 128
DEGW = 16


def _sc_edge_aggregate(with_deg):
    mesh = plsc.VectorSubcoreMesh(
        core_axis_name="c", subcore_axis_name="s", num_cores=NC, num_subcores=NS
    )
    outs = [jax.ShapeDtypeStruct((NC, N, D), jnp.float32)]
    scratch = [
        pltpu.VMEM_SHARED((N_ACC, D), jnp.float32),
        pltpu.VMEM((CPT, CH), jnp.int32),
        pltpu.VMEM((CPT, CH), jnp.int32),
        pltpu.VMEM((CH, D), jnp.float32),
        pltpu.VMEM((ZCH, D), jnp.float32),
        pltpu.SemaphoreType.DMA,
    ]
    if with_deg:
        outs.append(jax.ShapeDtypeStruct((NC, N, DEGW), jnp.float32))
        scratch += [
            pltpu.VMEM_SHARED((N_ACC, DEGW), jnp.float32),
            pltpu.VMEM((CH, DEGW), jnp.float32),
            pltpu.VMEM((ZCH, DEGW), jnp.float32),
        ]

    def body(h_hbm, src_hbm, dst_hbm, zd_hbm, z16_hbm, o16_hbm, *rest):
        if with_deg:
            (out_hbm, dout_hbm, acc, src_v, dst_v, rows_v, zv, gsem,
             dacc, ones_v, zv16) = rest
        else:
            out_hbm, acc, src_v, dst_v, rows_v, zv, gsem = rest
        cid = lax.axis_index("c")
        sid = lax.axis_index("s")
        w = cid * NS + sid

        pltpu.sync_copy(src_hbm.at[pl.ds(w * CPT, CPT)], src_v)
        pltpu.sync_copy(dst_hbm.at[pl.ds(w * CPT, CPT)], dst_v)
        pltpu.sync_copy(zd_hbm, zv)
        if with_deg:
            pltpu.sync_copy(z16_hbm, zv16)
            pltpu.sync_copy(o16_hbm, ones_v)

        zbase = sid * (N_ACC // NS)
        for z in range(N_ACC // NS // ZCH):
            pltpu.sync_copy(zv, acc.at[pl.ds(zbase + z * ZCH, ZCH)])
            if with_deg:
                pltpu.sync_copy(zv16, dacc.at[pl.ds(zbase + z * ZCH, ZCH)])
        plsc.subcore_barrier()

        def chunk(j, carry):
            pltpu.async_copy(h_hbm.at[src_v.at[j]], rows_v, gsem).wait()
            pltpu.sync_copy(rows_v, acc.at[dst_v.at[j]], add=True)
            if with_deg:
                pltpu.sync_copy(ones_v, dacc.at[dst_v.at[j]], add=True)
            return carry

        lax.fori_loop(0, CPT, chunk, 0)
        plsc.subcore_barrier()

        ob = sid * (N // NS)
        pltpu.sync_copy(acc.at[pl.ds(ob, N // NS)],
                        out_hbm.at[cid].at[pl.ds(ob, N // NS)])
        if with_deg:
            pltpu.sync_copy(dacc.at[pl.ds(ob, N // NS)],
                            dout_hbm.at[cid].at[pl.ds(ob, N // NS)])

    return pl.kernel(
        body,
        out_type=tuple(outs) if with_deg else outs[0],
        mesh=mesh,
        scratch_types=scratch,
    )


def _sage_body(p_ref, d_ref, h_ref, wl_ref, wr_ref, b_ref, o_ref):
    deg = jnp.maximum(d_ref[0, :, 0:1] + d_ref[1, :, 0:1], 1.0)
    agg = (p_ref[0] + p_ref[1]) / deg
    acc = (
        jnp.dot(agg, wl_ref[...], preferred_element_type=jnp.float32)
        + jnp.dot(h_ref[...], wr_ref[...], preferred_element_type=jnp.float32)
        + b_ref[...]
    )
    o_ref[...] = jnp.maximum(acc, 0.0)


_R = 1000


def _sage_tc(p, dsum, h, wl, wr, b2d):
    return pl.pallas_call(
        _sage_body,
        grid=(N // _R,),
        in_specs=[
            pl.BlockSpec((NC, _R, D), lambda i: (0, i, 0)),
            pl.BlockSpec((NC, _R, DEGW), lambda i: (0, i, 0)),
            pl.BlockSpec((_R, D), lambda i: (i, 0)),
            pl.BlockSpec((D, D), lambda i: (0, 0)),
            pl.BlockSpec((D, D), lambda i: (0, 0)),
            pl.BlockSpec((1, D), lambda i: (0, 0)),
        ],
        out_specs=pl.BlockSpec((_R, D), lambda i: (i, 0)),
        out_shape=jax.ShapeDtypeStruct((N, D), jnp.float32),
    )(p, dsum, h, wl, wr, b2d)


def _pool_body(h_ref, bt_ref, wg_ref, bg_ref, wl1_ref, bl1_ref, wl2_ref, bl2_ref, o_ref):
    h = h_ref[...]
    gate = jnp.dot(h, wg_ref[...], preferred_element_type=jnp.float32) + bg_ref[...]
    mask = bt_ref[...] == lax.broadcasted_iota(jnp.int32, (1, G), 1)
    gb = jnp.where(mask, gate, -jnp.inf)
    m = jnp.max(gb, axis=0, keepdims=True)
    m = jnp.where(jnp.isfinite(m), m, 0.0)
    e = jnp.where(mask, jnp.exp(gate - m), 0.0)
    s = jnp.sum(e, axis=0, keepdims=True)
    alpha = e / (s + 1e-16)
    g = lax.dot_general(alpha, h, (((0,), (0,)), ((), ())),
                        preferred_element_type=jnp.float32)
    g1 = jnp.maximum(
        jnp.dot(g, wl1_ref[...], preferred_element_type=jnp.float32) + bl1_ref[...],
        0.0,
    )
    out = jnp.dot(g1, wl2_ref[...], preferred_element_type=jnp.float32) + bl2_ref[...]
    mx = jnp.max(out, axis=1, keepdims=True)
    sh = out - mx
    lse = jnp.log(jnp.sum(jnp.exp(sh), axis=1, keepdims=True))
    o_ref[...] = sh - lse


def _pool_tc(h, bt, wg, bg2d, wl1, bl1_2d, wl2, bl2_2d):
    return pl.pallas_call(
        _pool_body,
        out_shape=jax.ShapeDtypeStruct((G, C), jnp.float32),
    )(h, bt, wg, bg2d, wl1, bl1_2d, wl2, bl2_2d)


_sc_agg_deg = _sc_edge_aggregate(True)
_sc_agg = _sc_edge_aggregate(False)


def kernel(x, edge_index, batch, W1_l, W1_r, b1, W2_l, W2_r, b2, W3_l, W3_r, b3,
           Wg, bg, Wl1, bl1, Wl2, bl2):
    src = edge_index[0]
    dst = edge_index[1]
    pad = E_PAD - E
    src2 = jnp.concatenate([src, jnp.zeros((pad,), jnp.int32)]).reshape(NW * CPT, CH)
    dst2 = jnp.concatenate([dst, jnp.full((pad,), N, jnp.int32)]).reshape(NW * CPT, CH)
    zd = jnp.zeros((ZCH, D), jnp.float32)
    z16 = jnp.zeros((ZCH, DEGW), jnp.float32)
    o16 = jnp.ones((CH, DEGW), jnp.float32)

    p1, degp = _sc_agg_deg(x, src2, dst2, zd, z16, o16)
    h1 = _sage_tc(p1, degp, x, W1_l, W1_r, b1.reshape(1, D))
    p2 = _sc_agg(h1, src2, dst2, zd, z16, o16)
    h2 = _sage_tc(p2, degp, h1, W2_l, W2_r, b2.reshape(1, H))
    p3 = _sc_agg(h2, src2, dst2, zd, z16, o16)
    h3 = _sage_tc(p3, degp, h2, W3_l, W3_r, b3.reshape(1, H))

    return _pool_tc(
        h3,
        batch.reshape(N, 1),
        Wg,
        bg.reshape(1, 1),
        Wl1,
        bl1.reshape(1, H),
        Wl2,
        bl2.reshape(1, C),
    )


# speedup vs baseline: 3.1159x; 3.1159x over previous
import functools

import jax
import jax.numpy as jnp
from jax import lax
from jax.experimental import pallas as pl
from jax.experimental.pallas import tpu as pltpu
from jax.experimental.pallas import tpu_sc as plsc

N = 10000
E = 320000
D = 128
H = 128
G = 64
C = 10

NC = 2
NS = 16
NW = NC * NS

CH = 128
CPT = 80
E_PAD = NW * CPT * CH
OB = 632
N_ACC = 10240
ZPT = N_ACC // NS
DEGW = 128


def _mesh():
    return plsc.VectorSubcoreMesh(
        core_axis_name="c", subcore_axis_name="s", num_cores=NC, num_subcores=NS
    )


def _copy_out(acc, out_hbm, cid, sid):
    tail = N - (NS - 1) * OB

    @pl.when(sid < NS - 1)
    def _():
        pltpu.sync_copy(acc.at[pl.ds(sid * OB, OB)],
                        out_hbm.at[cid].at[pl.ds(sid * OB, OB)])

    @pl.when(sid == NS - 1)
    def _():
        pltpu.sync_copy(acc.at[pl.ds((NS - 1) * OB, tail)],
                        out_hbm.at[cid].at[pl.ds((NS - 1) * OB, tail)])


@functools.lru_cache(maxsize=None)
def _sc_edge_aggregate():
    def body(h_hbm, src_hbm, dst_hbm, zd_hbm, out_hbm, acc, src_v, dst_v,
             rows_v, gsem):
        cid = lax.axis_index("c")
        sid = lax.axis_index("s")
        w = cid * NS + sid

        pltpu.sync_copy(src_hbm.at[pl.ds(w * CPT, CPT)], src_v)
        pltpu.sync_copy(dst_hbm.at[pl.ds(w * CPT, CPT)], dst_v)
        pltpu.sync_copy(zd_hbm, acc.at[pl.ds(sid * ZPT, ZPT)])
        plsc.subcore_barrier()

        def chunk(j, carry):
            pltpu.async_copy(h_hbm.at[src_v.at[j]], rows_v, gsem).wait()
            pltpu.sync_copy(rows_v, acc.at[dst_v.at[j]], add=True)
            return carry

        lax.fori_loop(0, CPT, chunk, 0)
        plsc.subcore_barrier()
        _copy_out(acc, out_hbm, cid, sid)

    return pl.kernel(
        body,
        out_type=jax.ShapeDtypeStruct((NC, N, D), jnp.float32),
        mesh=_mesh(),
        scratch_types=[
            pltpu.VMEM_SHARED((N_ACC, D), jnp.float32),
            pltpu.VMEM((CPT, CH), jnp.int32),
            pltpu.VMEM((CPT, CH), jnp.int32),
            pltpu.VMEM((CH, D), jnp.float32),
            pltpu.SemaphoreType.DMA,
        ],
    )


@functools.lru_cache(maxsize=None)
def _sc_degree():
    def body(dst_hbm, z16_hbm, o16_hbm, dout_hbm, dacc, dst_v, ones_v):
        cid = lax.axis_index("c")
        sid = lax.axis_index("s")
        w = cid * NS + sid

        pltpu.sync_copy(dst_hbm.at[pl.ds(w * CPT, CPT)], dst_v)
        pltpu.sync_copy(o16_hbm, ones_v)
        pltpu.sync_copy(z16_hbm, dacc.at[pl.ds(sid * ZPT, ZPT)])
        plsc.subcore_barrier()

        def chunk(j, carry):
            pltpu.sync_copy(ones_v, dacc.at[dst_v.at[j]], add=True)
            return carry

        lax.fori_loop(0, CPT, chunk, 0)
        plsc.subcore_barrier()
        _copy_out(dacc, dout_hbm, cid, sid)

    return pl.kernel(
        body,
        out_type=jax.ShapeDtypeStruct((NC, N, DEGW), jnp.float32),
        mesh=_mesh(),
        scratch_types=[
            pltpu.VMEM_SHARED((N_ACC, DEGW), jnp.float32),
            pltpu.VMEM((CPT, CH), jnp.int32),
            pltpu.VMEM((CH, DEGW), jnp.float32),
        ],
    )


def _sage_body(p_ref, d_ref, h_ref, wl_ref, wr_ref, b_ref, o_ref):
    deg = jnp.maximum(d_ref[0, :, 0:1] + d_ref[1, :, 0:1], 1.0)
    agg = (p_ref[0] + p_ref[1]) / deg
    acc = (
        jnp.dot(agg, wl_ref[...], preferred_element_type=jnp.float32)
        + jnp.dot(h_ref[...], wr_ref[...], preferred_element_type=jnp.float32)
        + b_ref[...]
    )
    o_ref[...] = jnp.maximum(acc, 0.0)


_R = 1000


def _sage_tc(p, dsum, h, wl, wr, b2d):
    return pl.pallas_call(
        _sage_body,
        grid=(N // _R,),
        in_specs=[
            pl.BlockSpec((NC, _R, D), lambda i: (0, i, 0)),
            pl.BlockSpec((NC, _R, DEGW), lambda i: (0, i, 0)),
            pl.BlockSpec((_R, D), lambda i: (i, 0)),
            pl.BlockSpec((D, D), lambda i: (0, 0)),
            pl.BlockSpec((D, D), lambda i: (0, 0)),
            pl.BlockSpec((1, D), lambda i: (0, 0)),
        ],
        out_specs=pl.BlockSpec((_R, D), lambda i: (i, 0)),
        out_shape=jax.ShapeDtypeStruct((N, D), jnp.float32),
    )(p, dsum, h, wl, wr, b2d)


def _pool_body(h_ref, bt_ref, wg_ref, bg_ref, wl1_ref, bl1_ref, wl2_ref, bl2_ref, o_ref):
    h = h_ref[...]
    gate = jnp.dot(h, wg_ref[...], preferred_element_type=jnp.float32) + bg_ref[...]
    mask = bt_ref[...] == lax.broadcasted_iota(jnp.int32, (1, G), 1)
    gb = jnp.where(mask, gate, -jnp.inf)
    m = jnp.max(gb, axis=0, keepdims=True)
    m = jnp.where(jnp.isfinite(m), m, 0.0)
    e = jnp.where(mask, jnp.exp(gate - m), 0.0)
    s = jnp.sum(e, axis=0, keepdims=True)
    alpha = e / (s + 1e-16)
    g = lax.dot_general(alpha, h, (((0,), (0,)), ((), ())),
                        preferred_element_type=jnp.float32)
    g1 = jnp.maximum(
        jnp.dot(g, wl1_ref[...], preferred_element_type=jnp.float32) + bl1_ref[...],
        0.0,
    )
    out = jnp.dot(g1, wl2_ref[...], preferred_element_type=jnp.float32) + bl2_ref[...]
    mx = jnp.max(out, axis=1, keepdims=True)
    sh = out - mx
    lse = jnp.log(jnp.sum(jnp.exp(sh), axis=1, keepdims=True))
    o_ref[...] = sh - lse


def _pool_tc(h, bt, wg, bg2d, wl1, bl1_2d, wl2, bl2_2d):
    return pl.pallas_call(
        _pool_body,
        out_shape=jax.ShapeDtypeStruct((G, C), jnp.float32),
    )(h, bt, wg, bg2d, wl1, bl1_2d, wl2, bl2_2d)


def kernel(x, edge_index, batch, W1_l, W1_r, b1, W2_l, W2_r, b2, W3_l, W3_r, b3,
           Wg, bg, Wl1, bl1, Wl2, bl2):
    src = edge_index[0]
    dst = edge_index[1]
    pad = E_PAD - E
    src2 = jnp.concatenate([src, jnp.zeros((pad,), jnp.int32)]).reshape(NW * CPT, CH)
    dst2 = jnp.concatenate([dst, jnp.full((pad,), N, jnp.int32)]).reshape(NW * CPT, CH)
    zd = jnp.zeros((ZPT, D), jnp.float32)
    z16 = jnp.zeros((ZPT, DEGW), jnp.float32)
    o16 = jnp.ones((CH, DEGW), jnp.float32)

    degp = _sc_degree()(dst2, z16, o16)
    p1 = _sc_edge_aggregate()(x, src2, dst2, zd)
    h1 = _sage_tc(p1, degp, x, W1_l, W1_r, b1.reshape(1, D))
    p2 = _sc_edge_aggregate()(h1, src2, dst2, zd)
    h2 = _sage_tc(p2, degp, h1, W2_l, W2_r, b2.reshape(1, H))
    p3 = _sc_edge_aggregate()(h2, src2, dst2, zd)
    h3 = _sage_tc(p3, degp, h2, W3_l, W3_r, b3.reshape(1, H))

    return _pool_tc(
        h3,
        batch.reshape(N, 1),
        Wg,
        bg.reshape(1, 1),
        Wl1,
        bl1.reshape(1, H),
        Wl2,
        bl2.reshape(1, C),
    )


# speedup vs baseline: 3.4429x; 1.1050x over previous
import functools

import jax
import jax.numpy as jnp
from jax import lax
from jax.experimental import pallas as pl
from jax.experimental.pallas import tpu as pltpu
from jax.experimental.pallas import tpu_sc as plsc

N = 10000
E = 320000
D = 128
H = 128
G = 64
C = 10

NC = 2
NS = 16
NW = NC * NS

CH = 128
CPT = 80
GRP = 8
NG = CPT // GRP
E_PAD = NW * CPT * CH
OB = 632
N_ACC = 10240
ZPT = N_ACC // NS
DEGW = 128


def _mesh():
    return plsc.VectorSubcoreMesh(
        core_axis_name="c", subcore_axis_name="s", num_cores=NC, num_subcores=NS
    )


def _copy_out(acc, out_hbm, cid, sid):
    tail = N - (NS - 1) * OB

    @pl.when(sid < NS - 1)
    def _():
        pltpu.sync_copy(acc.at[pl.ds(sid * OB, OB)],
                        out_hbm.at[cid].at[pl.ds(sid * OB, OB)])

    @pl.when(sid == NS - 1)
    def _():
        pltpu.sync_copy(acc.at[pl.ds((NS - 1) * OB, tail)],
                        out_hbm.at[cid].at[pl.ds((NS - 1) * OB, tail)])


@functools.lru_cache(maxsize=None)
def _sc_edge_aggregate():
    def body(h_hbm, src_hbm, dst_hbm, zd_hbm, out_hbm, acc, srcg, dst_v,
             rows_v, gsem0, gsem1, ssem0, ssem1, isem):
        gsems = (gsem0, gsem1)
        ssems = (ssem0, ssem1)
        cid = lax.axis_index("c")
        sid = lax.axis_index("s")
        w = cid * NS + sid

        pltpu.sync_copy(dst_hbm.at[pl.ds(w * CPT, CPT)], dst_v)
        pltpu.sync_copy(zd_hbm, acc.at[pl.ds(sid * ZPT, ZPT)])
        plsc.subcore_barrier()

        def stage(g, gb):
            pltpu.async_copy(src_hbm.at[pl.ds(w * CPT + g * GRP, GRP)],
                             srcg.at[gb], isem)

        def stage_wait(gb):
            pltpu.make_async_copy(src_hbm.at[pl.ds(0, GRP)], srcg.at[gb],
                                  isem).wait()

        def gather(gb, k, b):
            pltpu.async_copy(h_hbm.at[srcg.at[gb].at[k]], rows_v.at[b],
                             gsems[b])

        def gather_wait(b):
            pltpu.make_async_copy(h_hbm.at[pl.ds(0, CH)], rows_v.at[b],
                                  gsems[b]).wait()

        def scatter(j, b):
            pltpu.async_copy(rows_v.at[b], acc.at[dst_v.at[j]], ssems[b],
                             add=True)

        def scatter_wait(b):
            pltpu.make_async_copy(h_hbm.at[pl.ds(0, CH)], rows_v.at[b],
                                  ssems[b]).wait()

        stage(0, 0)

        def group(g, carry):
            gb = lax.rem(g, 2)
            stage_wait(gb)

            @pl.when(g + 1 < NG)
            def _():
                stage(g + 1, 1 - gb)

            gather(gb, 0, 0)

            for k in range(GRP):
                j = g * GRP + k
                b = k % 2
                nb = 1 - b

                @pl.when(j >= 1)
                def _():
                    scatter_wait(nb)

                if k + 1 < GRP:
                    gather(gb, k + 1, nb)
                gather_wait(b)
                scatter(j, b)
            return carry

        lax.fori_loop(0, NG, group, 0)
        scatter_wait(1)
        plsc.subcore_barrier()
        _copy_out(acc, out_hbm, cid, sid)

    return pl.kernel(
        body,
        out_type=jax.ShapeDtypeStruct((NC, N, D), jnp.float32),
        mesh=_mesh(),
        scratch_types=[
            pltpu.VMEM_SHARED((N_ACC, D), jnp.float32),
            pltpu.VMEM((2, GRP, CH), jnp.int32),
            pltpu.VMEM((CPT, CH), jnp.int32),
            pltpu.VMEM((2, CH, D), jnp.float32),
            pltpu.SemaphoreType.DMA,
            pltpu.SemaphoreType.DMA,
            pltpu.SemaphoreType.DMA,
            pltpu.SemaphoreType.DMA,
            pltpu.SemaphoreType.DMA,
        ],
    )


@functools.lru_cache(maxsize=None)
def _sc_degree():
    def body(dst_hbm, z16_hbm, o16_hbm, dout_hbm, dacc, dst_v, ones_v):
        cid = lax.axis_index("c")
        sid = lax.axis_index("s")
        w = cid * NS + sid

        pltpu.sync_copy(dst_hbm.at[pl.ds(w * CPT, CPT)], dst_v)
        pltpu.sync_copy(o16_hbm, ones_v)
        pltpu.sync_copy(z16_hbm, dacc.at[pl.ds(sid * ZPT, ZPT)])
        plsc.subcore_barrier()

        def chunk(j, carry):
            pltpu.sync_copy(ones_v, dacc.at[dst_v.at[j]], add=True)
            return carry

        lax.fori_loop(0, CPT, chunk, 0)
        plsc.subcore_barrier()
        _copy_out(dacc, dout_hbm, cid, sid)

    return pl.kernel(
        body,
        out_type=jax.ShapeDtypeStruct((NC, N, DEGW), jnp.float32),
        mesh=_mesh(),
        scratch_types=[
            pltpu.VMEM_SHARED((N_ACC, DEGW), jnp.float32),
            pltpu.VMEM((CPT, CH), jnp.int32),
            pltpu.VMEM((CH, DEGW), jnp.float32),
        ],
    )


def _sage_body(p_ref, d_ref, h_ref, wl_ref, wr_ref, b_ref, o_ref):
    deg = jnp.maximum(d_ref[0, :, 0:1] + d_ref[1, :, 0:1], 1.0)
    agg = (p_ref[0] + p_ref[1]) / deg
    acc = (
        jnp.dot(agg, wl_ref[...], preferred_element_type=jnp.float32)
        + jnp.dot(h_ref[...], wr_ref[...], preferred_element_type=jnp.float32)
        + b_ref[...]
    )
    o_ref[...] = jnp.maximum(acc, 0.0)


_R = 1000


def _sage_tc(p, dsum, h, wl, wr, b2d):
    return pl.pallas_call(
        _sage_body,
        grid=(N // _R,),
        in_specs=[
            pl.BlockSpec((NC, _R, D), lambda i: (0, i, 0)),
            pl.BlockSpec((NC, _R, DEGW), lambda i: (0, i, 0)),
            pl.BlockSpec((_R, D), lambda i: (i, 0)),
            pl.BlockSpec((D, D), lambda i: (0, 0)),
            pl.BlockSpec((D, D), lambda i: (0, 0)),
            pl.BlockSpec((1, D), lambda i: (0, 0)),
        ],
        out_specs=pl.BlockSpec((_R, D), lambda i: (i, 0)),
        out_shape=jax.ShapeDtypeStruct((N, D), jnp.float32),
    )(p, dsum, h, wl, wr, b2d)


def _pool_body(h_ref, bt_ref, wg_ref, bg_ref, wl1_ref, bl1_ref, wl2_ref, bl2_ref, o_ref):
    h = h_ref[...]
    gate = jnp.dot(h, wg_ref[...], preferred_element_type=jnp.float32) + bg_ref[...]
    mask = bt_ref[...] == lax.broadcasted_iota(jnp.int32, (1, G), 1)
    gb = jnp.where(mask, gate, -jnp.inf)
    m = jnp.max(gb, axis=0, keepdims=True)
    m = jnp.where(jnp.isfinite(m), m, 0.0)
    e = jnp.where(mask, jnp.exp(gate - m), 0.0)
    s = jnp.sum(e, axis=0, keepdims=True)
    alpha = e / (s + 1e-16)
    g = lax.dot_general(alpha, h, (((0,), (0,)), ((), ())),
                        preferred_element_type=jnp.float32)
    g1 = jnp.maximum(
        jnp.dot(g, wl1_ref[...], preferred_element_type=jnp.float32) + bl1_ref[...],
        0.0,
    )
    out = jnp.dot(g1, wl2_ref[...], preferred_element_type=jnp.float32) + bl2_ref[...]
    mx = jnp.max(out, axis=1, keepdims=True)
    sh = out - mx
    lse = jnp.log(jnp.sum(jnp.exp(sh), axis=1, keepdims=True))
    o_ref[...] = sh - lse


def _pool_tc(h, bt, wg, bg2d, wl1, bl1_2d, wl2, bl2_2d):
    return pl.pallas_call(
        _pool_body,
        out_shape=jax.ShapeDtypeStruct((G, C), jnp.float32),
    )(h, bt, wg, bg2d, wl1, bl1_2d, wl2, bl2_2d)


def kernel(x, edge_index, batch, W1_l, W1_r, b1, W2_l, W2_r, b2, W3_l, W3_r, b3,
           Wg, bg, Wl1, bl1, Wl2, bl2):
    src = edge_index[0]
    dst = edge_index[1]
    pad = E_PAD - E
    src2 = jnp.concatenate([src, jnp.zeros((pad,), jnp.int32)]).reshape(NW * CPT, CH)
    dst2 = jnp.concatenate([dst, jnp.full((pad,), N, jnp.int32)]).reshape(NW * CPT, CH)
    zd = jnp.zeros((ZPT, D), jnp.float32)
    z16 = jnp.zeros((ZPT, DEGW), jnp.float32)
    o16 = jnp.ones((CH, DEGW), jnp.float32)

    degp = _sc_degree()(dst2, z16, o16)
    p1 = _sc_edge_aggregate()(x, src2, dst2, zd)
    h1 = _sage_tc(p1, degp, x, W1_l, W1_r, b1.reshape(1, D))
    p2 = _sc_edge_aggregate()(h1, src2, dst2, zd)
    h2 = _sage_tc(p2, degp, h1, W2_l, W2_r, b2.reshape(1, H))
    p3 = _sc_edge_aggregate()(h2, src2, dst2, zd)
    h3 = _sage_tc(p3, degp, h2, W3_l, W3_r, b3.reshape(1, H))

    return _pool_tc(
        h3,
        batch.reshape(N, 1),
        Wg,
        bg.reshape(1, 1),
        Wl1,
        bl1.reshape(1, H),
        Wl2,
        bl2.reshape(1, C),
    )


# speedup vs baseline: 11.1884x; 3.2497x over previous
import functools

import jax
import jax.numpy as jnp
from jax import lax
from jax.experimental import pallas as pl
from jax.experimental.pallas import tpu as pltpu
from jax.experimental.pallas import tpu_sc as plsc

N = 10000
E = 320000
D = 128
H = 128
G = 64
C = 10

NC = 2
NS = 16
NW = NC * NS

CH = 128
CPT = 80
GRP = 8
NG = CPT // GRP
E_PAD = NW * CPT * CH
OB = 632
N_ACC = 10240
ZPT = N_ACC // NS
DEGW = 128


def _mesh():
    return plsc.VectorSubcoreMesh(
        core_axis_name="c", subcore_axis_name="s", num_cores=NC, num_subcores=NS
    )


def _copy_out(acc, out_hbm, cid, sid):
    tail = N - (NS - 1) * OB

    @pl.when(sid < NS - 1)
    def _():
        pltpu.sync_copy(acc.at[pl.ds(sid * OB, OB)],
                        out_hbm.at[cid].at[pl.ds(sid * OB, OB)])

    @pl.when(sid == NS - 1)
    def _():
        pltpu.sync_copy(acc.at[pl.ds((NS - 1) * OB, tail)],
                        out_hbm.at[cid].at[pl.ds((NS - 1) * OB, tail)])


@functools.lru_cache(maxsize=None)
def _sc_edge_aggregate():
    def body(h_hbm, src_hbm, dst_hbm, zd_hbm, out_hbm, acc, srcg, dst_v,
             rows_v, gsem0, gsem1, ssem0, ssem1, isem):
        gsems = (gsem0, gsem1)
        ssems = (ssem0, ssem1)
        cid = lax.axis_index("c")
        sid = lax.axis_index("s")
        w = cid * NS + sid

        pltpu.sync_copy(dst_hbm.at[pl.ds(w * CPT, CPT)], dst_v)
        pltpu.sync_copy(zd_hbm, acc.at[pl.ds(sid * ZPT, ZPT)])
        plsc.subcore_barrier()

        def stage(g, gb):
            pltpu.async_copy(src_hbm.at[pl.ds(w * CPT + g * GRP, GRP)],
                             srcg.at[gb], isem)

        def stage_wait(gb):
            pltpu.make_async_copy(src_hbm.at[pl.ds(0, GRP)], srcg.at[gb],
                                  isem).wait()

        def gather(gb, k, b):
            pltpu.async_copy(h_hbm.at[srcg.at[gb].at[k]], rows_v.at[b],
                             gsems[b])

        def gather_wait(b):
            pltpu.make_async_copy(h_hbm.at[pl.ds(0, CH)], rows_v.at[b],
                                  gsems[b]).wait()

        def scatter(j, b):
            pltpu.async_copy(rows_v.at[b], acc.at[dst_v.at[j]], ssems[b],
                             add=True)

        def scatter_wait(b):
            pltpu.make_async_copy(h_hbm.at[pl.ds(0, CH)], rows_v.at[b],
                                  ssems[b]).wait()

        stage(0, 0)

        def group(g, carry):
            gb = lax.rem(g, 2)
            stage_wait(gb)

            @pl.when(g + 1 < NG)
            def _():
                stage(g + 1, 1 - gb)

            gather(gb, 0, 0)

            for k in range(GRP):
                j = g * GRP + k
                b = k % 2
                nb = 1 - b

                @pl.when(j >= 1)
                def _():
                    scatter_wait(nb)

                if k + 1 < GRP:
                    gather(gb, k + 1, nb)
                gather_wait(b)
                scatter(j, b)
            return carry

        lax.fori_loop(0, NG, group, 0)
        scatter_wait(1)
        plsc.subcore_barrier()
        _copy_out(acc, out_hbm, cid, sid)

    return pl.kernel(
        body,
        out_type=jax.ShapeDtypeStruct((NC, N, D), jnp.float32),
        mesh=_mesh(),
        scratch_types=[
            pltpu.VMEM_SHARED((N_ACC, D), jnp.float32),
            pltpu.VMEM((2, GRP, CH), jnp.int32),
            pltpu.VMEM((CPT, CH), jnp.int32),
            pltpu.VMEM((2, CH, D), jnp.float32),
            pltpu.SemaphoreType.DMA,
            pltpu.SemaphoreType.DMA,
            pltpu.SemaphoreType.DMA,
            pltpu.SemaphoreType.DMA,
            pltpu.SemaphoreType.DMA,
        ],
    )


@functools.lru_cache(maxsize=None)
def _sc_degree():
    def body(dst_hbm, z16_hbm, o16_hbm, dout_hbm, dacc, dst_v, ones_v):
        cid = lax.axis_index("c")
        sid = lax.axis_index("s")
        w = cid * NS + sid

        pltpu.sync_copy(dst_hbm.at[pl.ds(w * CPT, CPT)], dst_v)
        pltpu.sync_copy(o16_hbm, ones_v)
        pltpu.sync_copy(z16_hbm, dacc.at[pl.ds(sid * ZPT, ZPT)])
        plsc.subcore_barrier()

        def chunk(j, carry):
            pltpu.sync_copy(ones_v, dacc.at[dst_v.at[j]], add=True)
            return carry

        lax.fori_loop(0, CPT, chunk, 0)
        plsc.subcore_barrier()
        _copy_out(dacc, dout_hbm, cid, sid)

    return pl.kernel(
        body,
        out_type=jax.ShapeDtypeStruct((NC, N, DEGW), jnp.float32),
        mesh=_mesh(),
        scratch_types=[
            pltpu.VMEM_SHARED((N_ACC, DEGW), jnp.float32),
            pltpu.VMEM((CPT, CH), jnp.int32),
            pltpu.VMEM((CH, DEGW), jnp.float32),
        ],
    )


def _sage_body(p_ref, d_ref, h_ref, wl_ref, wr_ref, b_ref, o_ref):
    deg = jnp.maximum(d_ref[0, :, 0:1] + d_ref[1, :, 0:1], 1.0)
    agg = (p_ref[0] + p_ref[1]) / deg
    acc = (
        jnp.dot(agg, wl_ref[...], preferred_element_type=jnp.float32)
        + jnp.dot(h_ref[...], wr_ref[...], preferred_element_type=jnp.float32)
        + b_ref[...]
    )
    o_ref[...] = jnp.maximum(acc, 0.0)


_R = 1000


def _sage_tc(p, dsum, h, wl, wr, b2d):
    return pl.pallas_call(
        _sage_body,
        grid=(N // _R,),
        in_specs=[
            pl.BlockSpec((NC, _R, D), lambda i: (0, i, 0)),
            pl.BlockSpec((NC, _R, DEGW), lambda i: (0, i, 0)),
            pl.BlockSpec((_R, D), lambda i: (i, 0)),
            pl.BlockSpec((D, D), lambda i: (0, 0)),
            pl.BlockSpec((D, D), lambda i: (0, 0)),
            pl.BlockSpec((1, D), lambda i: (0, 0)),
        ],
        out_specs=pl.BlockSpec((_R, D), lambda i: (i, 0)),
        out_shape=jax.ShapeDtypeStruct((N, D), jnp.float32),
    )(p, dsum, h, wl, wr, b2d)


def _pool_body(h_ref, bt_ref, wg_ref, bg_ref, wl1_ref, bl1_ref, wl2_ref, bl2_ref, o_ref):
    h = h_ref[...]
    gate = jnp.dot(h, wg_ref[...], preferred_element_type=jnp.float32) + bg_ref[...]
    mask = bt_ref[...] == lax.broadcasted_iota(jnp.int32, (1, G), 1)
    gb = jnp.where(mask, gate, -jnp.inf)
    m = jnp.max(gb, axis=0, keepdims=True)
    m = jnp.where(jnp.isfinite(m), m, 0.0)
    e = jnp.where(mask, jnp.exp(gate - m), 0.0)
    s = jnp.sum(e, axis=0, keepdims=True)
    alpha = e / (s + 1e-16)
    g = lax.dot_general(alpha, h, (((0,), (0,)), ((), ())),
                        preferred_element_type=jnp.float32)
    g1 = jnp.maximum(
        jnp.dot(g, wl1_ref[...], preferred_element_type=jnp.float32) + bl1_ref[...],
        0.0,
    )
    out = jnp.dot(g1, wl2_ref[...], preferred_element_type=jnp.float32) + bl2_ref[...]
    mx = jnp.max(out, axis=1, keepdims=True)
    sh = out - mx
    lse = jnp.log(jnp.sum(jnp.exp(sh), axis=1, keepdims=True))
    o_ref[...] = sh - lse


def _pool_tc(h, bt, wg, bg2d, wl1, bl1_2d, wl2, bl2_2d):
    return pl.pallas_call(
        _pool_body,
        out_shape=jax.ShapeDtypeStruct((G, C), jnp.float32),
    )(h, bt, wg, bg2d, wl1, bl1_2d, wl2, bl2_2d)


def kernel(x, edge_index, batch, W1_l, W1_r, b1, W2_l, W2_r, b2, W3_l, W3_r, b3,
           Wg, bg, Wl1, bl1, Wl2, bl2):
    ept = E // NW
    padt = CPT * CH - ept
    src_r = edge_index[0].reshape(NW, ept)
    dst_r = edge_index[1].reshape(NW, ept)
    fill = jax.lax.broadcasted_iota(jnp.int32, (NW, padt), 1)
    src2 = jnp.concatenate([src_r, fill % N], axis=1).reshape(NW * CPT, CH)
    dst2 = jnp.concatenate([dst_r, N + fill % (N_ACC - N)], axis=1).reshape(NW * CPT, CH)
    zd = jnp.zeros((ZPT, D), jnp.float32)
    z16 = jnp.zeros((ZPT, DEGW), jnp.float32)
    o16 = jnp.ones((CH, DEGW), jnp.float32)

    degp = _sc_degree()(dst2, z16, o16)
    p1 = _sc_edge_aggregate()(x, src2, dst2, zd)
    h1 = _sage_tc(p1, degp, x, W1_l, W1_r, b1.reshape(1, D))
    p2 = _sc_edge_aggregate()(h1, src2, dst2, zd)
    h2 = _sage_tc(p2, degp, h1, W2_l, W2_r, b2.reshape(1, H))
    p3 = _sc_edge_aggregate()(h2, src2, dst2, zd)
    h3 = _sage_tc(p3, degp, h2, W3_l, W3_r, b3.reshape(1, H))

    return _pool_tc(
        h3,
        batch.reshape(N, 1),
        Wg,
        bg.reshape(1, 1),
        Wl1,
        bl1.reshape(1, H),
        Wl2,
        bl2.reshape(1, C),
    )


# speedup vs baseline: 11.6807x; 1.0440x over previous
import functools

import jax
import jax.numpy as jnp
from jax import lax
from jax.experimental import pallas as pl
from jax.experimental.pallas import tpu as pltpu
from jax.experimental.pallas import tpu_sc as plsc

N = 10000
E = 320000
D = 128
H = 128
G = 64
C = 10

NC = 2
NS = 16
NW = NC * NS

CH = 80
CPT = 128
GRP = 8
NG = CPT // GRP
NBUF = 4
E_PAD = NW * CPT * CH
OB = 632
N_ACC = 10112
ZPT = N_ACC // NS
DEGW = 128


def _mesh():
    return plsc.VectorSubcoreMesh(
        core_axis_name="c", subcore_axis_name="s", num_cores=NC, num_subcores=NS
    )


def _copy_out(acc, out_hbm, cid, sid):
    tail = N - (NS - 1) * OB

    @pl.when(sid < NS - 1)
    def _():
        pltpu.sync_copy(acc.at[pl.ds(sid * OB, OB)],
                        out_hbm.at[cid].at[pl.ds(sid * OB, OB)])

    @pl.when(sid == NS - 1)
    def _():
        pltpu.sync_copy(acc.at[pl.ds((NS - 1) * OB, tail)],
                        out_hbm.at[cid].at[pl.ds((NS - 1) * OB, tail)])


@functools.lru_cache(maxsize=None)
def _sc_edge_aggregate():
    def body(h_hbm, src_hbm, dst_hbm, zd_hbm, out_hbm, acc, srcg, dstg,
             rows_v, gsem0, gsem1, gsem2, gsem3, ssem0, ssem1, ssem2, ssem3,
             isem):
        gsems = (gsem0, gsem1, gsem2, gsem3)
        ssems = (ssem0, ssem1, ssem2, ssem3)
        cid = lax.axis_index("c")
        sid = lax.axis_index("s")
        w = cid * NS + sid

        pltpu.sync_copy(zd_hbm, acc.at[pl.ds(sid * ZPT, ZPT)])

        def stage(g, gb):
            pltpu.async_copy(src_hbm.at[pl.ds(w * CPT + g * GRP, GRP)],
                             srcg.at[gb], isem)
            pltpu.async_copy(dst_hbm.at[pl.ds(w * CPT + g * GRP, GRP)],
                             dstg.at[gb], isem)

        def stage_wait(gb):
            pltpu.make_async_copy(src_hbm.at[pl.ds(0, GRP)], srcg.at[gb],
                                  isem).wait()
            pltpu.make_async_copy(dst_hbm.at[pl.ds(0, GRP)], dstg.at[gb],
                                  isem).wait()

        def gather(gb, k, b):
            pltpu.async_copy(h_hbm.at[srcg.at[gb].at[k]], rows_v.at[b],
                             gsems[b])

        def gather_wait(b):
            pltpu.make_async_copy(h_hbm.at[pl.ds(0, CH)], rows_v.at[b],
                                  gsems[b]).wait()

        def scatter(gb, k, b):
            pltpu.async_copy(rows_v.at[b], acc.at[dstg.at[gb].at[k]],
                             ssems[b], add=True)

        def scatter_wait(b):
            pltpu.make_async_copy(h_hbm.at[pl.ds(0, CH)], rows_v.at[b],
                                  ssems[b]).wait()

        plsc.subcore_barrier()

        stage(0, 0)
        stage_wait(0)
        stage(1, 1)
        for k in range(3):
            gather(0, k, k)

        def group(g, carry):
            gb = lax.rem(g, 2)
            ngb = 1 - gb
            for k in range(GRP):
                b = k % NBUF
                j = g * GRP + k
                gather_wait(b)
                scatter(gb, k, b)

                @pl.when(j >= 1)
                def _():
                    scatter_wait((k + 3) % NBUF)

                if k == 0:
                    @pl.when(jnp.logical_and(g >= 1, g + 1 < NG))
                    def _():
                        stage(g + 1, ngb)

                if k == 5:
                    @pl.when(g + 1 < NG)
                    def _():
                        stage_wait(ngb)

                if k < 5:
                    @pl.when(j + 3 < CPT)
                    def _():
                        gather(gb, k + 3, (k + 3) % NBUF)
                else:
                    @pl.when(j + 3 < CPT)
                    def _():
                        gather(ngb, k - 5, (k + 3) % NBUF)
            return carry

        lax.fori_loop(0, NG, group, 0)
        scatter_wait((CPT - 1) % NBUF)
        plsc.subcore_barrier()
        _copy_out(acc, out_hbm, cid, sid)

    return pl.kernel(
        body,
        out_type=jax.ShapeDtypeStruct((NC, N, D), jnp.float32),
        mesh=_mesh(),
        scratch_types=[
            pltpu.VMEM_SHARED((N_ACC, D), jnp.float32),
            pltpu.VMEM((2, GRP, CH), jnp.int32),
            pltpu.VMEM((2, GRP, CH), jnp.int32),
            pltpu.VMEM((NBUF, CH, D), jnp.float32),
            pltpu.SemaphoreType.DMA,
            pltpu.SemaphoreType.DMA,
            pltpu.SemaphoreType.DMA,
            pltpu.SemaphoreType.DMA,
            pltpu.SemaphoreType.DMA,
            pltpu.SemaphoreType.DMA,
            pltpu.SemaphoreType.DMA,
            pltpu.SemaphoreType.DMA,
            pltpu.SemaphoreType.DMA,
        ],
    )


@functools.lru_cache(maxsize=None)
def _sc_degree():
    def body(dst_hbm, z16_hbm, o16_hbm, dout_hbm, dacc, dst_v, ones_v):
        cid = lax.axis_index("c")
        sid = lax.axis_index("s")
        w = cid * NS + sid

        pltpu.sync_copy(dst_hbm.at[pl.ds(w * CPT, CPT)], dst_v)
        pltpu.sync_copy(o16_hbm, ones_v)
        pltpu.sync_copy(z16_hbm, dacc.at[pl.ds(sid * ZPT, ZPT)])
        plsc.subcore_barrier()

        def chunk(j, carry):
            pltpu.sync_copy(ones_v, dacc.at[dst_v.at[j]], add=True)
            return carry

        lax.fori_loop(0, CPT, chunk, 0)
        plsc.subcore_barrier()
        _copy_out(dacc, dout_hbm, cid, sid)

    return pl.kernel(
        body,
        out_type=jax.ShapeDtypeStruct((NC, N, DEGW), jnp.float32),
        mesh=_mesh(),
        scratch_types=[
            pltpu.VMEM_SHARED((N_ACC, DEGW), jnp.float32),
            pltpu.VMEM((CPT, CH), jnp.int32),
            pltpu.VMEM((CH, DEGW), jnp.float32),
        ],
    )


def _sage_body(p_ref, d_ref, h_ref, wl_ref, wr_ref, b_ref, o_ref):
    deg = jnp.maximum(d_ref[0, :, 0:1] + d_ref[1, :, 0:1], 1.0)
    agg = (p_ref[0] + p_ref[1]) / deg
    acc = (
        jnp.dot(agg, wl_ref[...], preferred_element_type=jnp.float32)
        + jnp.dot(h_ref[...], wr_ref[...], preferred_element_type=jnp.float32)
        + b_ref[...]
    )
    o_ref[...] = jnp.maximum(acc, 0.0)


_R = 1000


def _sage_tc(p, dsum, h, wl, wr, b2d):
    return pl.pallas_call(
        _sage_body,
        grid=(N // _R,),
        in_specs=[
            pl.BlockSpec((NC, _R, D), lambda i: (0, i, 0)),
            pl.BlockSpec((NC, _R, DEGW), lambda i: (0, i, 0)),
            pl.BlockSpec((_R, D), lambda i: (i, 0)),
            pl.BlockSpec((D, D), lambda i: (0, 0)),
            pl.BlockSpec((D, D), lambda i: (0, 0)),
            pl.BlockSpec((1, D), lambda i: (0, 0)),
        ],
        out_specs=pl.BlockSpec((_R, D), lambda i: (i, 0)),
        out_shape=jax.ShapeDtypeStruct((N, D), jnp.float32),
    )(p, dsum, h, wl, wr, b2d)


def _pool_body(h_ref, bt_ref, wg_ref, bg_ref, wl1_ref, bl1_ref, wl2_ref, bl2_ref, o_ref):
    h = h_ref[...]
    gate = jnp.dot(h, wg_ref[...], preferred_element_type=jnp.float32) + bg_ref[...]
    mask = bt_ref[...] == lax.broadcasted_iota(jnp.int32, (1, G), 1)
    gb = jnp.where(mask, gate, -jnp.inf)
    m = jnp.max(gb, axis=0, keepdims=True)
    m = jnp.where(jnp.isfinite(m), m, 0.0)
    e = jnp.where(mask, jnp.exp(gate - m), 0.0)
    s = jnp.sum(e, axis=0, keepdims=True)
    alpha = e / (s + 1e-16)
    g = lax.dot_general(alpha, h, (((0,), (0,)), ((), ())),
                        preferred_element_type=jnp.float32)
    g1 = jnp.maximum(
        jnp.dot(g, wl1_ref[...], preferred_element_type=jnp.float32) + bl1_ref[...],
        0.0,
    )
    out = jnp.dot(g1, wl2_ref[...], preferred_element_type=jnp.float32) + bl2_ref[...]
    mx = jnp.max(out, axis=1, keepdims=True)
    sh = out - mx
    lse = jnp.log(jnp.sum(jnp.exp(sh), axis=1, keepdims=True))
    o_ref[...] = sh - lse


def _pool_tc(h, bt, wg, bg2d, wl1, bl1_2d, wl2, bl2_2d):
    return pl.pallas_call(
        _pool_body,
        out_shape=jax.ShapeDtypeStruct((G, C), jnp.float32),
    )(h, bt, wg, bg2d, wl1, bl1_2d, wl2, bl2_2d)


def kernel(x, edge_index, batch, W1_l, W1_r, b1, W2_l, W2_r, b2, W3_l, W3_r, b3,
           Wg, bg, Wl1, bl1, Wl2, bl2):
    ept = E // NW
    padt = CPT * CH - ept
    src_r = edge_index[0].reshape(NW, ept)
    dst_r = edge_index[1].reshape(NW, ept)
    fill = jax.lax.broadcasted_iota(jnp.int32, (NW, padt), 1)
    src2 = jnp.concatenate([src_r, fill % N], axis=1).reshape(NW * CPT, CH)
    dst2 = jnp.concatenate([dst_r, N + fill % (N_ACC - N)], axis=1).reshape(NW * CPT, CH)
    zd = jnp.zeros((ZPT, D), jnp.float32)
    z16 = jnp.zeros((ZPT, DEGW), jnp.float32)
    o16 = jnp.ones((CH, DEGW), jnp.float32)

    degp = _sc_degree()(dst2, z16, o16)
    p1 = _sc_edge_aggregate()(x, src2, dst2, zd)
    h1 = _sage_tc(p1, degp, x, W1_l, W1_r, b1.reshape(1, D))
    p2 = _sc_edge_aggregate()(h1, src2, dst2, zd)
    h2 = _sage_tc(p2, degp, h1, W2_l, W2_r, b2.reshape(1, H))
    p3 = _sc_edge_aggregate()(h2, src2, dst2, zd)
    h3 = _sage_tc(p3, degp, h2, W3_l, W3_r, b3.reshape(1, H))

    return _pool_tc(
        h3,
        batch.reshape(N, 1),
        Wg,
        bg.reshape(1, 1),
        Wl1,
        bl1.reshape(1, H),
        Wl2,
        bl2.reshape(1, C),
    )


# speedup vs baseline: 13.2899x; 1.1378x over previous
import functools

import jax
import jax.numpy as jnp
from jax import lax
from jax.experimental import pallas as pl
from jax.experimental.pallas import tpu as pltpu
from jax.experimental.pallas import tpu_sc as plsc

N = 10000
E = 320000
D = 128
H = 128
G = 64
C = 10

NC = 2
NS = 16
NW = NC * NS

CH = 80
CPT = 128
GRP = 8
NG = CPT // GRP
NBUF = 4
E_PAD = NW * CPT * CH
OB = 632
N_ACC = 10112
ZPT = N_ACC // NS


def _mesh():
    return plsc.VectorSubcoreMesh(
        core_axis_name="c", subcore_axis_name="s", num_cores=NC, num_subcores=NS
    )


def _copy_out(acc, out_hbm, cid, sid):
    tail = N - (NS - 1) * OB

    @pl.when(sid < NS - 1)
    def _():
        pltpu.sync_copy(acc.at[pl.ds(sid * OB, OB)],
                        out_hbm.at[cid].at[pl.ds(sid * OB, OB)])

    @pl.when(sid == NS - 1)
    def _():
        pltpu.sync_copy(acc.at[pl.ds((NS - 1) * OB, tail)],
                        out_hbm.at[cid].at[pl.ds((NS - 1) * OB, tail)])


@functools.lru_cache(maxsize=None)
def _sc_edge_aggregate():
    def body(h_hbm, src_hbm, dst_hbm, zd_hbm, out_hbm, acc, srcg, dstg,
             rows_v, gsem0, gsem1, gsem2, gsem3, ssem0, ssem1, ssem2, ssem3,
             isem):
        gsems = (gsem0, gsem1, gsem2, gsem3)
        ssems = (ssem0, ssem1, ssem2, ssem3)
        cid = lax.axis_index("c")
        sid = lax.axis_index("s")
        w = cid * NS + sid

        pltpu.sync_copy(zd_hbm, acc.at[pl.ds(sid * ZPT, ZPT)])

        def stage(g, gb):
            pltpu.async_copy(src_hbm.at[pl.ds(w * CPT + g * GRP, GRP)],
                             srcg.at[gb], isem)
            pltpu.async_copy(dst_hbm.at[pl.ds(w * CPT + g * GRP, GRP)],
                             dstg.at[gb], isem)

        def stage_wait(gb):
            pltpu.make_async_copy(src_hbm.at[pl.ds(0, GRP)], srcg.at[gb],
                                  isem).wait()
            pltpu.make_async_copy(dst_hbm.at[pl.ds(0, GRP)], dstg.at[gb],
                                  isem).wait()

        def gather(gb, k, b):
            pltpu.async_copy(h_hbm.at[srcg.at[gb].at[k]], rows_v.at[b],
                             gsems[b])

        def gather_wait(b):
            pltpu.make_async_copy(h_hbm.at[pl.ds(0, CH)], rows_v.at[b],
                                  gsems[b]).wait()

        def scatter(gb, k, b):
            pltpu.async_copy(rows_v.at[b], acc.at[dstg.at[gb].at[k]],
                             ssems[b], add=True)

        def scatter_wait(b):
            pltpu.make_async_copy(h_hbm.at[pl.ds(0, CH)], rows_v.at[b],
                                  ssems[b]).wait()

        plsc.subcore_barrier()

        stage(0, 0)
        stage_wait(0)
        stage(1, 1)
        for k in range(3):
            gather(0, k, k)

        def group(g, carry):
            gb = lax.rem(g, 2)
            ngb = 1 - gb
            for k in range(GRP):
                b = k % NBUF
                j = g * GRP + k
                gather_wait(b)
                scatter(gb, k, b)

                @pl.when(j >= 1)
                def _():
                    scatter_wait((k + 3) % NBUF)

                if k == 0:
                    @pl.when(jnp.logical_and(g >= 1, g + 1 < NG))
                    def _():
                        stage(g + 1, ngb)

                if k == 5:
                    @pl.when(g + 1 < NG)
                    def _():
                        stage_wait(ngb)

                if k < 5:
                    @pl.when(j + 3 < CPT)
                    def _():
                        gather(gb, k + 3, (k + 3) % NBUF)
                else:
                    @pl.when(j + 3 < CPT)
                    def _():
                        gather(ngb, k - 5, (k + 3) % NBUF)
            return carry

        lax.fori_loop(0, NG, group, 0)
        scatter_wait((CPT - 1) % NBUF)
        plsc.subcore_barrier()
        _copy_out(acc, out_hbm, cid, sid)

    return pl.kernel(
        body,
        out_type=jax.ShapeDtypeStruct((NC, N, D), jnp.float32),
        mesh=_mesh(),
        scratch_types=[
            pltpu.VMEM_SHARED((N_ACC, D), jnp.float32),
            pltpu.VMEM((2, GRP, CH), jnp.int32),
            pltpu.VMEM((2, GRP, CH), jnp.int32),
            pltpu.VMEM((NBUF, CH, D), jnp.float32),
            pltpu.SemaphoreType.DMA,
            pltpu.SemaphoreType.DMA,
            pltpu.SemaphoreType.DMA,
            pltpu.SemaphoreType.DMA,
            pltpu.SemaphoreType.DMA,
            pltpu.SemaphoreType.DMA,
            pltpu.SemaphoreType.DMA,
            pltpu.SemaphoreType.DMA,
            pltpu.SemaphoreType.DMA,
        ],
    )


N_HIST = 10240
_RED = N_HIST // NS


@functools.lru_cache(maxsize=None)
def _sc_degree():
    def body(dst_hbm, dout_hbm, shared, dst_v, hist, red, res):
        cid = lax.axis_index("c")
        sid = lax.axis_index("s")
        w = cid * NS + sid

        pltpu.sync_copy(dst_hbm.at[pl.ds(w * CPT * CH, CPT * CH)], dst_v)

        zeros16 = jnp.zeros((16,), jnp.float32)
        ones16 = jnp.ones((16,), jnp.float32)

        def zstep(i, carry):
            hist[pl.ds(i * 16, 16)] = zeros16
            return carry

        lax.fori_loop(0, N_HIST // 16, zstep, 0)

        mask16 = jnp.ones((16,), jnp.bool_)

        def count(i, carry):
            idx = dst_v[pl.ds(i * 16, 16)]
            plsc.addupdate_scatter(hist, [idx], ones16, mask=mask16)
            return carry

        lax.fori_loop(0, CPT * CH // 16, count, 0)

        pltpu.sync_copy(hist, shared.at[pl.ds(sid * N_HIST, N_HIST)])
        plsc.subcore_barrier()
        for r in range(NS):
            pltpu.sync_copy(shared.at[pl.ds(r * N_HIST + sid * _RED, _RED)],
                            red.at[pl.ds(r * _RED, _RED)])

        def rstep(c, carry):
            s = jnp.zeros((16,), jnp.float32)
            for r in range(NS):
                s = s + red[pl.ds(r * _RED + c * 16, 16)]
            res[pl.ds(c * 16, 16)] = s
            return carry

        lax.fori_loop(0, _RED // 16, rstep, 0)

        tail = N - (NS - 1) * _RED

        @pl.when(sid < NS - 1)
        def _():
            pltpu.sync_copy(res.at[pl.ds(0, _RED)],
                            dout_hbm.at[pl.ds(cid * N + sid * _RED, _RED)])

        @pl.when(sid == NS - 1)
        def _():
            pltpu.sync_copy(res.at[pl.ds(0, tail)],
                            dout_hbm.at[pl.ds(cid * N + (NS - 1) * _RED, tail)])

    return pl.kernel(
        body,
        out_type=jax.ShapeDtypeStruct((NC * N,), jnp.float32),
        mesh=_mesh(),
        compiler_params=pltpu.CompilerParams(needs_layout_passes=False),
        scratch_types=[
            pltpu.VMEM_SHARED((NS * N_HIST,), jnp.float32),
            pltpu.VMEM((CPT * CH,), jnp.int32),
            pltpu.VMEM((N_HIST,), jnp.float32),
            pltpu.VMEM((NS * _RED,), jnp.float32),
            pltpu.VMEM((_RED,), jnp.float32),
        ],
    )


def _sage_body(p_ref, d_ref, h_ref, wl_ref, wr_ref, b_ref, o_ref):
    deg = jnp.maximum(d_ref[0] + d_ref[1], 1.0)
    agg = (p_ref[0] + p_ref[1]) / deg
    acc = (
        jnp.dot(agg, wl_ref[...], preferred_element_type=jnp.float32)
        + jnp.dot(h_ref[...], wr_ref[...], preferred_element_type=jnp.float32)
        + b_ref[...]
    )
    o_ref[...] = jnp.maximum(acc, 0.0)


_R = 1000


def _sage_tc(p, dsum, h, wl, wr, b2d):
    return pl.pallas_call(
        _sage_body,
        grid=(N // _R,),
        in_specs=[
            pl.BlockSpec((NC, _R, D), lambda i: (0, i, 0)),
            pl.BlockSpec((NC, _R, 1), lambda i: (0, i, 0)),
            pl.BlockSpec((_R, D), lambda i: (i, 0)),
            pl.BlockSpec((D, D), lambda i: (0, 0)),
            pl.BlockSpec((D, D), lambda i: (0, 0)),
            pl.BlockSpec((1, D), lambda i: (0, 0)),
        ],
        out_specs=pl.BlockSpec((_R, D), lambda i: (i, 0)),
        out_shape=jax.ShapeDtypeStruct((N, D), jnp.float32),
    )(p, dsum, h, wl, wr, b2d)


def _pool_body(h_ref, bt_ref, wg_ref, bg_ref, wl1_ref, bl1_ref, wl2_ref, bl2_ref, o_ref):
    h = h_ref[...]
    gate = jnp.dot(h, wg_ref[...], preferred_element_type=jnp.float32) + bg_ref[...]
    mask = bt_ref[...] == lax.broadcasted_iota(jnp.int32, (1, G), 1)
    gb = jnp.where(mask, gate, -jnp.inf)
    m = jnp.max(gb, axis=0, keepdims=True)
    m = jnp.where(jnp.isfinite(m), m, 0.0)
    e = jnp.where(mask, jnp.exp(gate - m), 0.0)
    s = jnp.sum(e, axis=0, keepdims=True)
    alpha = e / (s + 1e-16)
    g = lax.dot_general(alpha, h, (((0,), (0,)), ((), ())),
                        preferred_element_type=jnp.float32)
    g1 = jnp.maximum(
        jnp.dot(g, wl1_ref[...], preferred_element_type=jnp.float32) + bl1_ref[...],
        0.0,
    )
    out = jnp.dot(g1, wl2_ref[...], preferred_element_type=jnp.float32) + bl2_ref[...]
    mx = jnp.max(out, axis=1, keepdims=True)
    sh = out - mx
    lse = jnp.log(jnp.sum(jnp.exp(sh), axis=1, keepdims=True))
    o_ref[...] = sh - lse


def _pool_tc(h, bt, wg, bg2d, wl1, bl1_2d, wl2, bl2_2d):
    return pl.pallas_call(
        _pool_body,
        out_shape=jax.ShapeDtypeStruct((G, C), jnp.float32),
    )(h, bt, wg, bg2d, wl1, bl1_2d, wl2, bl2_2d)


def kernel(x, edge_index, batch, W1_l, W1_r, b1, W2_l, W2_r, b2, W3_l, W3_r, b3,
           Wg, bg, Wl1, bl1, Wl2, bl2):
    ept = E // NW
    padt = CPT * CH - ept
    src_r = edge_index[0].reshape(NW, ept)
    dst_r = edge_index[1].reshape(NW, ept)
    fill = jax.lax.broadcasted_iota(jnp.int32, (NW, padt), 1)
    src2 = jnp.concatenate([src_r, fill % N], axis=1).reshape(NW * CPT, CH)
    dst2 = jnp.concatenate([dst_r, N + fill % (N_ACC - N)], axis=1).reshape(NW * CPT, CH)
    zd = jnp.zeros((ZPT, D), jnp.float32)

    degp = _sc_degree()(dst2.reshape(-1)).reshape(NC, N)[:, :, None]
    p1 = _sc_edge_aggregate()(x, src2, dst2, zd)
    h1 = _sage_tc(p1, degp, x, W1_l, W1_r, b1.reshape(1, D))
    p2 = _sc_edge_aggregate()(h1, src2, dst2, zd)
    h2 = _sage_tc(p2, degp, h1, W2_l, W2_r, b2.reshape(1, H))
    p3 = _sc_edge_aggregate()(h2, src2, dst2, zd)
    h3 = _sage_tc(p3, degp, h2, W3_l, W3_r, b3.reshape(1, H))

    return _pool_tc(
        h3,
        batch.reshape(N, 1),
        Wg,
        bg.reshape(1, 1),
        Wl1,
        bl1.reshape(1, H),
        Wl2,
        bl2.reshape(1, C),
    )
